# baseline (speedup 1.0000x reference)
import functools

import jax
import jax.numpy as jnp
from jax import lax
from jax.experimental import pallas as pl
from jax.experimental.pallas import tpu as pltpu
from jax.experimental.pallas import tpu_sc as plsc

NC = 2
NS = 16
L = 16
NW = NC * NS
B = 16384
D = 128
D1 = 50
BPW = B // NW
GI = 32
NG = BPW // GI

_mesh = plsc.VectorSubcoreMesh(core_axis_name="c", subcore_axis_name="s")


@functools.partial(
    pl.kernel,
    out_type=jax.ShapeDtypeStruct((B, D), jnp.float32),
    mesh=_mesh,
    compiler_params=pltpu.CompilerParams(needs_layout_passes=False),
    scratch_types=[
        pltpu.VMEM((BPW,), jnp.int32),
        pltpu.VMEM((NG, GI, D), jnp.float32),
        pltpu.VMEM_SHARED((100, D), jnp.float32),
        [pltpu.SemaphoreType.DMA] * NG,
        pltpu.SemaphoreType.DMA,
        pltpu.SemaphoreType.DMA,
    ],
)
def _sc_emb(z_hbm, tab_hbm, emb_hbm, idx_v, rows_v, tab_sh, gsems, osem, ssem):
    wid = lax.axis_index("s") * NC + lax.axis_index("c")
    base = wid * BPW
    sid = lax.axis_index("s")

    idx_d = pltpu.async_copy(z_hbm.at[pl.ds(base, BPW)], idx_v, ssem)

    @pl.when(sid == 0)
    def _():
        pltpu.sync_copy(tab_hbm, tab_sh)

    idx_d.wait()
    plsc.subcore_barrier()
    gd = [pltpu.async_copy(tab_sh.at[idx_v.at[pl.ds(j * GI, GI)]],
                           rows_v.at[j], gsems[j])
          for j in range(NG)]
    od = []
    for j in range(NG):
        gd[j].wait()
        od.append(pltpu.async_copy(rows_v.at[j],
                                   emb_hbm.at[pl.ds(base + j * GI, GI)], osem))
    for o in od:
        o.wait()


TC_R = 4096


def _tc_lin_body(z1_ref, z2_ref, z3_ref, w1_ref, b1_ref, w2_ref, b2_ref,
                 w3_ref, b3_ref, o1t_ref, o2_ref, o3_ref):
    z2 = z2_ref[...].reshape(TC_R, 1)
    z3 = z3_ref[...].reshape(TC_R, 1)
    z1 = z1_ref[...].reshape(1, TC_R)
    o1t_ref[...] = (w1_ref[...].reshape(D1, 1) * z1
                    + b1_ref[...].reshape(D1, 1))
    o2_ref[...] = z2 * w2_ref[...].reshape(1, D) + b2_ref[...].reshape(1, D)
    o3_ref[...] = z3 * w3_ref[...].reshape(1, D) + b3_ref[...].reshape(1, D)


_tc_lin = pl.pallas_call(
    _tc_lin_body,
    grid=(B // TC_R,),
    in_specs=[
        pl.BlockSpec((TC_R,), lambda i: (i,)),
        pl.BlockSpec((TC_R,), lambda i: (i,)),
        pl.BlockSpec((TC_R,), lambda i: (i,)),
        pl.BlockSpec((D1,), lambda i: (0,)),
        pl.BlockSpec((D1,), lambda i: (0,)),
        pl.BlockSpec((D,), lambda i: (0,)),
        pl.BlockSpec((D,), lambda i: (0,)),
        pl.BlockSpec((D,), lambda i: (0,)),
        pl.BlockSpec((D,), lambda i: (0,)),
    ],
    out_specs=[
        pl.BlockSpec((D1, TC_R), lambda i: (0, i)),
        pl.BlockSpec((TC_R, D), lambda i: (i, 0)),
        pl.BlockSpec((TC_R, D), lambda i: (i, 0)),
    ],
    out_shape=[
        jax.ShapeDtypeStruct((D1, B), jnp.float32),
        jax.ShapeDtypeStruct((B, D), jnp.float32),
        jax.ShapeDtypeStruct((B, D), jnp.float32),
    ],
)


def kernel(z, z1, z2, z3, emb_table, W1, b1, W2, b2, W3, b3):
    emb = _sc_emb(z.astype(jnp.int32), emb_table)
    o1t, o2, o3 = _tc_lin(z1, z2, z3, W1.reshape(-1), b1,
                          W2.reshape(-1), b2, W3.reshape(-1), b3)
    return emb, o1t.T, o2, o3


# speedup vs baseline: 1.0203x; 1.0203x over previous
import functools

import jax
import jax.numpy as jnp
from jax import lax
from jax.experimental import pallas as pl
from jax.experimental.pallas import tpu as pltpu
from jax.experimental.pallas import tpu_sc as plsc

NC = 2
NS = 16
L = 16
NW = NC * NS
B = 16384
D = 128
D1 = 50
BPW = B // NW
GI = 64
NG = BPW // GI

_mesh = plsc.VectorSubcoreMesh(core_axis_name="c", subcore_axis_name="s")


@functools.partial(
    pl.kernel,
    out_type=jax.ShapeDtypeStruct((B, D), jnp.float32),
    mesh=_mesh,
    compiler_params=pltpu.CompilerParams(needs_layout_passes=False),
    scratch_types=[
        pltpu.VMEM((BPW,), jnp.int32),
        pltpu.VMEM((NG, GI, D), jnp.float32),
        pltpu.VMEM_SHARED((100, D), jnp.float32),
        [pltpu.SemaphoreType.DMA] * NG,
        pltpu.SemaphoreType.DMA,
        pltpu.SemaphoreType.DMA,
    ],
)
def _sc_emb(z_hbm, tab_hbm, emb_hbm, idx_v, rows_v, tab_sh, gsems, osem, ssem):
    wid = lax.axis_index("s") * NC + lax.axis_index("c")
    base = wid * BPW
    sid = lax.axis_index("s")

    idx_d = pltpu.async_copy(z_hbm.at[pl.ds(base, BPW)], idx_v, ssem)

    @pl.when(sid == 0)
    def _():
        pltpu.sync_copy(tab_hbm, tab_sh)

    idx_d.wait()
    plsc.subcore_barrier()
    gd = [pltpu.async_copy(tab_sh.at[idx_v.at[pl.ds(j * GI, GI)]],
                           rows_v.at[j], gsems[j])
          for j in range(NG)]
    od = []
    for j in range(NG):
        gd[j].wait()
        od.append(pltpu.async_copy(rows_v.at[j],
                                   emb_hbm.at[pl.ds(base + j * GI, GI)], osem))
    for o in od:
        o.wait()


TC_R = 4096


def _tc_lin_body(z1_ref, z2_ref, z3_ref, w1_ref, b1_ref, w2_ref, b2_ref,
                 w3_ref, b3_ref, o1t_ref, o2_ref, o3_ref):
    z2 = z2_ref[...].reshape(TC_R, 1)
    z3 = z3_ref[...].reshape(TC_R, 1)
    z1 = z1_ref[...].reshape(1, TC_R)
    o1t_ref[...] = (w1_ref[...].reshape(D1, 1) * z1
                    + b1_ref[...].reshape(D1, 1))
    o2_ref[...] = z2 * w2_ref[...].reshape(1, D) + b2_ref[...].reshape(1, D)
    o3_ref[...] = z3 * w3_ref[...].reshape(1, D) + b3_ref[...].reshape(1, D)


_tc_lin = pl.pallas_call(
    _tc_lin_body,
    grid=(B // TC_R,),
    in_specs=[
        pl.BlockSpec((TC_R,), lambda i: (i,)),
        pl.BlockSpec((TC_R,), lambda i: (i,)),
        pl.BlockSpec((TC_R,), lambda i: (i,)),
        pl.BlockSpec((D1,), lambda i: (0,)),
        pl.BlockSpec((D1,), lambda i: (0,)),
        pl.BlockSpec((D,), lambda i: (0,)),
        pl.BlockSpec((D,), lambda i: (0,)),
        pl.BlockSpec((D,), lambda i: (0,)),
        pl.BlockSpec((D,), lambda i: (0,)),
    ],
    out_specs=[
        pl.BlockSpec((D1, TC_R), lambda i: (0, i)),
        pl.BlockSpec((TC_R, D), lambda i: (i, 0)),
        pl.BlockSpec((TC_R, D), lambda i: (i, 0)),
    ],
    out_shape=[
        jax.ShapeDtypeStruct((D1, B), jnp.float32),
        jax.ShapeDtypeStruct((B, D), jnp.float32),
        jax.ShapeDtypeStruct((B, D), jnp.float32),
    ],
)


def kernel(z, z1, z2, z3, emb_table, W1, b1, W2, b2, W3, b3):
    emb = _sc_emb(z.astype(jnp.int32), emb_table)
    o1t, o2, o3 = _tc_lin(z1, z2, z3, W1.reshape(-1), b1,
                          W2.reshape(-1), b2, W3.reshape(-1), b3)
    return emb, o1t.T, o2, o3
